# baseline (speedup 1.0000x reference)
import functools
import math

import jax
import jax.numpy as jnp
from jax import lax
from jax.experimental import pallas as pl
from jax.experimental.pallas import tpu as pltpu
from jax.experimental.pallas import tpu_sc as plsc

_NC = 2
_NS = 16
_L = 16

_S = 30000
_E_NUM = 19000
_K_NUM = 1000
_N = _S + _E_NUM + _K_NUM
_D = 64
_LEAKY = 0.8

_MM_KW = dict(preferred_element_type=jnp.float32,
              precision=lax.Precision.HIGHEST)

_SC_PARAMS = pltpu.CompilerParams(needs_layout_passes=False,
                                  use_tc_tiling_on_sc=False)


@functools.lru_cache(maxsize=None)
def _make_spmm(n_nodes, n_edges, d):
    half = n_nodes // 2
    assert n_nodes % 2 == 0
    zrows = 56
    rpt = math.ceil((half + 1) / (_NS * zrows)) * zrows
    acc_rows = _NS * rpt
    dummy = acc_rows - 1
    wb_full = half // rpt
    wb_rem = half - wb_full * rpt

    ept = n_edges // _NS
    assert ept * _NS == n_edges
    ch = 400
    assert ept % ch == 0 and ch % _L == 0
    n_ch = ept // ch
    groups = ch // _L
    nsb_cap = (ch + 255) // 128
    csplit = d // _L
    rb = 2

    def body(src_hbm, dst_hbm, val_hbm, emb_hbm, out_hbm,
             esrc, edst, evals, fsrc, fdst, fval, rows, sidx, zbuf, acc,
             esem, gsem, ssem):
        core = lax.axis_index("c")
        sid = lax.axis_index("s")
        lo = core * half

        @pl.loop(0, zrows)
        def _(r):
            for c in range(csplit):
                zbuf[r, pl.ds(c * _L, _L)] = jnp.zeros((_L,), jnp.float32)

        @pl.loop(0, rpt // zrows)
        def _(k):
            pltpu.sync_copy(zbuf, acc.at[pl.ds(sid * rpt + k * zrows, zrows)])

        plsc.subcore_barrier()

        ebase = sid * ept

        def edge_prefetch(ci, b):
            base = ebase + ci * ch
            pltpu.async_copy(src_hbm.at[pl.ds(base, ch)], esrc.at[b],
                             esem.at[b])
            pltpu.async_copy(dst_hbm.at[pl.ds(base, ch)], edst.at[b],
                             esem.at[b])
            pltpu.async_copy(val_hbm.at[pl.ds(base, ch)], evals.at[b],
                             esem.at[b])

        def edge_wait(b):
            for _ in range(3):
                pltpu.make_async_copy(src_hbm.at[pl.ds(0, ch)], esrc.at[b],
                                      esem.at[b]).wait()

        edge_prefetch(0, 0)

        def chunk(ci, q):
            b = lax.rem(ci, 2)

            @pl.when(ci + 1 < n_ch)
            def _():
                edge_prefetch(ci + 1, lax.rem(ci + 1, 2))

            edge_wait(b)

            def filt(g, cur):
                s = esrc[b, pl.ds(g * _L, _L)]
                dl = edst[b, pl.ds(g * _L, _L)] - lo
                v = evals[b, pl.ds(g * _L, _L)]
                m = (dl >= 0) & (dl < half)
                mi = m.astype(jnp.int32)
                pos = cur + plsc.cumsum(mi) - 1
                r = lax.shift_right_logical(pos, 7)
                cc = lax.bitwise_and(pos, 127)
                plsc.store_scatter(fsrc, [r, cc], s, mask=m)
                plsc.store_scatter(fdst, [r, cc], dl, mask=m)
                plsc.store_scatter(fval, [r, cc], v, mask=m)
                return cur + jnp.sum(mi)

            cur = lax.fori_loop(0, groups, filt, jnp.int32(0), unroll=2)

            iot = lax.iota(jnp.int32, _L)
            for k in range(128 // _L):
                pos = cur + (k * _L) + iot
                r = lax.shift_right_logical(pos, 7)
                cc = lax.bitwise_and(pos, 127)
                plsc.store_scatter(fsrc, [r, cc], jnp.zeros((_L,), jnp.int32))
                plsc.store_scatter(fdst, [r, cc],
                                   jnp.full((_L,), dummy, jnp.int32))
                plsc.store_scatter(fval, [r, cc], jnp.zeros((_L,), jnp.float32))

            n_sub = lax.shift_right_logical(cur + 127, 7)

            def sub(j, carry):
                @pl.when(j < n_sub)
                def _():
                    qj = q + j
                    s = lax.rem(qj, rb)

                    @pl.when(qj >= rb)
                    def _():
                        pltpu.make_async_copy(emb_hbm.at[pl.ds(0, 128)],
                                              rows.at[s], ssem.at[s]).wait()

                    pltpu.async_copy(emb_hbm.at[fsrc.at[j]], rows.at[s],
                                     gsem.at[s])

                @pl.when(j >= 1)
                def _():
                    jm = j - 1
                    sp = lax.rem(q + jm, rb)
                    pltpu.make_async_copy(emb_hbm.at[fsrc.at[jm]],
                                          rows.at[sp], gsem.at[sp]).wait()

                    @pl.loop(0, 128 // _L)
                    def _(g):
                        sl16 = pl.ds(g * _L, _L)
                        vvec = fval[jm, sl16]
                        sidx[sp, sl16] = fdst[jm, sl16]
                        for i in range(_L):
                            e = g * _L + i
                            vs = vvec[i]
                            for c in range(csplit):
                                sl = pl.ds(c * _L, _L)
                                rows[sp, e, sl] = rows[sp, e, sl] * vs

                    pltpu.async_copy(rows.at[sp], acc.at[sidx.at[sp]],
                                     ssem.at[sp], add=True)
                return carry

            lax.fori_loop(0, n_sub + 1, sub, jnp.int32(0))
            return q + n_sub

        qf = lax.fori_loop(0, n_ch, chunk, jnp.int32(0))

        def drain(k, carry):
            s = lax.rem(qf - 1 - k, rb)
            pltpu.make_async_copy(emb_hbm.at[pl.ds(0, 128)], rows.at[s],
                                  ssem.at[s]).wait()
            return carry

        lax.fori_loop(0, jnp.minimum(qf, rb), drain, jnp.int32(0))

        plsc.subcore_barrier()

        @pl.when(sid < wb_full)
        def _():
            pltpu.sync_copy(acc.at[pl.ds(sid * rpt, rpt)],
                            out_hbm.at[pl.ds(lo + sid * rpt, rpt)])

        if wb_rem:
            @pl.when(sid == wb_full)
            def _():
                pltpu.sync_copy(acc.at[pl.ds(wb_full * rpt, wb_rem)],
                                out_hbm.at[pl.ds(lo + wb_full * rpt, wb_rem)])

    mesh = plsc.VectorSubcoreMesh(core_axis_name="c", subcore_axis_name="s")
    return pl.kernel(
        body,
        out_type=jax.ShapeDtypeStruct((n_nodes, d), jnp.float32),
        mesh=mesh,
        compiler_params=_SC_PARAMS,
        scratch_types=[
            pltpu.VMEM((2, ch), jnp.int32),
            pltpu.VMEM((2, ch), jnp.int32),
            pltpu.VMEM((2, ch), jnp.float32),
            pltpu.VMEM((nsb_cap, 128), jnp.int32),
            pltpu.VMEM((nsb_cap, 128), jnp.int32),
            pltpu.VMEM((nsb_cap, 128), jnp.float32),
            pltpu.VMEM((rb, 128, d), jnp.float32),
            pltpu.VMEM((rb, 128), jnp.int32),
            pltpu.VMEM((zrows, d), jnp.float32),
            pltpu.VMEM_SHARED((acc_rows, d), jnp.float32),
            pltpu.SemaphoreType.DMA((2,)),
            pltpu.SemaphoreType.DMA((rb,)),
            pltpu.SemaphoreType.DMA((rb,)),
        ],
    )


def _combine(ae, a1, a2, a3, r1, r2, r3, w1, w2, w3, Wc, bc, Wc1, bc1):
    n, d = ae.shape
    blk = 1000
    assert n % blk == 0

    def body(aer, a1r, a2r, a3r, r1r, r2r, r3r, w1r, w2r, w3r,
             wcr, bcr, wc1r, bc1r, o):
        base = aer[...]
        hol = (base + a1r[...] + a2r[...] + a3r[...]) * 0.25
        rgt = (base + r1r[...] + r2r[...] + r3r[...]) * 0.25
        wrg = (base + w1r[...] + w2r[...] + w3r[...]) * 0.25
        wc = wcr[...]
        h = (jnp.dot(rgt, wc[:d], **_MM_KW) + jnp.dot(wrg, wc[d:], **_MM_KW)
             + bcr[...])
        dis = jnp.where(h >= 0, h, h * _LEAKY)
        wc1 = wc1r[...]
        h2 = (jnp.dot(dis, wc1[:d], **_MM_KW) + jnp.dot(hol, wc1[d:], **_MM_KW)
              + bc1r[...])
        o[...] = jnp.where(h2 >= 0, h2, h2 * _LEAKY)

    emb_spec = pl.BlockSpec((blk, d), lambda i: (i, 0))
    w_spec = pl.BlockSpec((2 * d, d), lambda i: (0, 0))
    b_spec = pl.BlockSpec((1, d), lambda i: (0, 0))
    return pl.pallas_call(
        body,
        grid=(n // blk,),
        in_specs=[emb_spec] * 10 + [w_spec, b_spec, w_spec, b_spec],
        out_specs=pl.BlockSpec((blk, d), lambda i: (i, 0)),
        out_shape=jax.ShapeDtypeStruct((n, d), jnp.float32),
    )(ae, a1, a2, a3, r1, r2, r3, w1, w2, w3,
      Wc, bc.reshape(1, d), Wc1, bc1.reshape(1, d))


def _make_batch_gather(n_nodes, d, batch, disc_n):
    per = batch // (_NC * _NS)
    assert per * _NC * _NS == batch and per % 128 == 0
    nseg = per // 128

    def body(emb_hbm, sid_hbm, eid_hbm, disc_hbm,
             stus_hbm, exers_hbm, disc_out_hbm,
             idx, rows, dvec, dout):
        core = lax.axis_index("c")
        s = lax.axis_index("s")
        w = s * _NC + core
        base = w * per

        pltpu.sync_copy(sid_hbm.at[pl.ds(base, per)], idx)
        for k in range(nseg):
            pltpu.sync_copy(emb_hbm.at[idx.at[pl.ds(k * 128, 128)]],
                            rows.at[pl.ds(k * 128, 128)])
        pltpu.sync_copy(rows, stus_hbm.at[pl.ds(base, per)])

        pltpu.sync_copy(eid_hbm.at[pl.ds(base, per)], idx)
        pltpu.sync_copy(disc_hbm, dvec)

        @pl.loop(0, per // _L)
        def _(g):
            sl = pl.ds(g * _L, _L)
            e = idx[sl]
            dout[sl] = plsc.load_gather(dvec, [e])
            idx[sl] = e + _S

        pltpu.sync_copy(dout, disc_out_hbm.at[pl.ds(base, per)])
        for k in range(nseg):
            pltpu.sync_copy(emb_hbm.at[idx.at[pl.ds(k * 128, 128)]],
                            rows.at[pl.ds(k * 128, 128)])
        pltpu.sync_copy(rows, exers_hbm.at[pl.ds(base, per)])

    mesh = plsc.VectorSubcoreMesh(core_axis_name="c", subcore_axis_name="s")
    return pl.kernel(
        body,
        out_type=(jax.ShapeDtypeStruct((batch, d), jnp.float32),
                  jax.ShapeDtypeStruct((batch, d), jnp.float32),
                  jax.ShapeDtypeStruct((batch,), jnp.float32)),
        mesh=mesh,
        compiler_params=_SC_PARAMS,
        scratch_types=[
            pltpu.VMEM((per,), jnp.int32),
            pltpu.VMEM((per, d), jnp.float32),
            pltpu.VMEM((disc_n,), jnp.float32),
            pltpu.VMEM((per,), jnp.float32),
        ],
    )


def _project(x, w, b):
    bsz, d = x.shape
    k = w.shape[1]
    blk = min(bsz, 1024)
    assert bsz % blk == 0

    def body(xr, wr, br, o):
        o[...] = jnp.dot(xr[...], wr[...], **_MM_KW) + br[...]

    return pl.pallas_call(
        body,
        grid=(bsz // blk,),
        in_specs=[pl.BlockSpec((blk, d), lambda i: (i, 0)),
                  pl.BlockSpec((d, k), lambda i: (0, 0)),
                  pl.BlockSpec((1, k), lambda i: (0, 0))],
        out_specs=pl.BlockSpec((blk, k), lambda i: (i, 0)),
        out_shape=jax.ShapeDtypeStruct((bsz, k), jnp.float32),
    )(x, w, b.reshape(1, k))


def kernel(student_id, exercise_id, q_mask, all_src, all_dst, all_val,
           right_src, right_dst, right_val, wrong_src, wrong_dst, wrong_val,
           stu_emb, exer_emb, know_emb, disc_emb, Wc, bc, Wc1, bc1,
           Wts, bts, Wte, bte, Wtk, btk):
    all_emb = jnp.concatenate([stu_emb, exer_emb, know_emb], axis=0)
    n_edges = all_src.shape[0]
    spmm = _make_spmm(_N, n_edges, _D)

    layer_outs = []
    for src, dst, val in ((all_src, all_dst, all_val),
                          (right_src, right_dst, right_val),
                          (wrong_src, wrong_dst, wrong_val)):
        src = src.astype(jnp.int32)
        dst = dst.astype(jnp.int32)
        cur = all_emb
        for _ in range(3):
            cur = spmm(src, dst, val, cur)
            layer_outs.append(cur)

    out_embs = _combine(all_emb, *layer_outs, Wc, bc, Wc1, bc1)

    gather = _make_batch_gather(_N, _D, student_id.shape[0], _E_NUM)
    stus_rows, exers_rows, disc_flat = gather(
        out_embs, student_id.astype(jnp.int32), exercise_id.astype(jnp.int32),
        disc_emb.reshape(-1))

    bsf = _project(stus_rows, Wts, bts)
    bef = _project(exers_rows, Wte, bte)
    kf = _project(out_embs[_S + _E_NUM:], Wtk, btk)
    return bsf, bef, disc_flat.reshape(-1, 1), kf


# speedup vs baseline: 5.0558x; 5.0558x over previous
import functools
import math

import jax
import jax.numpy as jnp
from jax import lax
from jax.experimental import pallas as pl
from jax.experimental.pallas import tpu as pltpu
from jax.experimental.pallas import tpu_sc as plsc

_NC = 2
_NS = 16
_L = 16

_S = 30000
_E_NUM = 19000
_K_NUM = 1000
_N = _S + _E_NUM + _K_NUM
_D = 64
_LEAKY = 0.8

_MM_KW = dict(preferred_element_type=jnp.float32,
              precision=lax.Precision.HIGHEST)

_SC_PARAMS = pltpu.CompilerParams(needs_layout_passes=False,
                                  use_tc_tiling_on_sc=False)


@functools.lru_cache(maxsize=None)
def _make_spmm(n_nodes, n_edges, d):
    half = n_nodes // 2
    assert n_nodes % 2 == 0
    rpt = math.ceil((half + 1) / (_NS * 16)) * 16
    acc_rows = _NS * rpt
    dummy = acc_rows - 1
    wb_full = half // rpt
    wb_rem = half - wb_full * rpt
    zfull, zrem = divmod(rpt, 128)

    ept = n_edges // _NS
    assert ept * _NS == n_edges
    ch = 2000
    assert ept % ch == 0 and ch % _L == 0
    n_ch = ept // ch
    groups = ch // _L
    nsb_cap = (ch + 255) // 128
    csplit = d // _L

    def body(src_hbm, dst_hbm, val_hbm, emb_hbm, out_hbm,
             esrc, edst, evals, fsrc, fdst, fval, rows, sidx, acc,
             dsem, gsem0, gsem1, ssem0, ssem1):
        core = lax.axis_index("c")
        sid = lax.axis_index("s")
        lo = core * half

        @pl.loop(0, 128)
        def _(r):
            for c in range(csplit):
                rows[0, r, pl.ds(c * _L, _L)] = jnp.zeros((_L,), jnp.float32)

        abase = sid * rpt
        z128 = rows.at[0]
        zpart = rows.at[0].at[pl.ds(0, max(zrem, 1))]
        for k in range(zfull):
            pltpu.async_copy(z128, acc.at[pl.ds(abase + k * 128, 128)], dsem)
        if zrem:
            pltpu.async_copy(zpart,
                             acc.at[pl.ds(abase + zfull * 128, zrem)], dsem)
        for k in range(zfull):
            pltpu.make_async_copy(z128, acc.at[pl.ds(abase, 128)], dsem).wait()
        if zrem:
            pltpu.make_async_copy(zpart, acc.at[pl.ds(abase, zrem)],
                                  dsem).wait()

        plsc.subcore_barrier()

        def gissue(s, qj, j):
            rslot = rows.at[s]
            gsm = gsem0 if s == 0 else gsem1
            ssm = ssem0 if s == 0 else ssem1

            @pl.when(qj >= 2)
            def _():
                pltpu.make_async_copy(emb_hbm.at[pl.ds(0, 128)], rslot,
                                      ssm).wait()

            pltpu.async_copy(emb_hbm.at[fsrc.at[j]], rslot, gsm)

        def gfinish(s, jm):
            rslot = rows.at[s]
            gsm = gsem0 if s == 0 else gsem1
            ssm = ssem0 if s == 0 else ssem1
            pltpu.make_async_copy(emb_hbm.at[fsrc.at[jm]], rslot, gsm).wait()

            @pl.loop(0, 128 // _L)
            def _(g):
                sl16 = pl.ds(g * _L, _L)
                vvec = fval[jm, sl16]
                sidx[s, sl16] = fdst[jm, sl16]
                for i in range(_L):
                    e = g * _L + i
                    vs = vvec[i]
                    for c in range(csplit):
                        sl = pl.ds(c * _L, _L)
                        rows[s, e, sl] = rows[s, e, sl] * vs

            pltpu.async_copy(rslot, acc.at[sidx.at[s]], ssm, add=True)

        ebase = sid * ept

        def chunk(ci, q):
            base = ebase + ci * ch
            pltpu.sync_copy(src_hbm.at[pl.ds(base, ch)], esrc)
            pltpu.sync_copy(dst_hbm.at[pl.ds(base, ch)], edst)
            pltpu.sync_copy(val_hbm.at[pl.ds(base, ch)], evals)

            def filt(g, cur):
                s = esrc[pl.ds(g * _L, _L)]
                dl = edst[pl.ds(g * _L, _L)] - lo
                v = evals[pl.ds(g * _L, _L)]
                m = (dl >= 0) & (dl < half)
                mi = m.astype(jnp.int32)
                pos = cur + plsc.cumsum(mi) - 1
                r = lax.shift_right_logical(pos, 7)
                cc = lax.bitwise_and(pos, 127)
                plsc.store_scatter(fsrc, [r, cc], s, mask=m)
                plsc.store_scatter(fdst, [r, cc], dl, mask=m)
                plsc.store_scatter(fval, [r, cc], v, mask=m)
                return cur + jnp.sum(mi)

            cur = lax.fori_loop(0, groups, filt, jnp.int32(0), unroll=2)

            iot = lax.iota(jnp.int32, _L)
            for k in range(128 // _L):
                pos = cur + (k * _L) + iot
                r = lax.shift_right_logical(pos, 7)
                cc = lax.bitwise_and(pos, 127)
                plsc.store_scatter(fsrc, [r, cc], jnp.zeros((_L,), jnp.int32))
                plsc.store_scatter(fdst, [r, cc],
                                   jnp.full((_L,), dummy, jnp.int32))
                plsc.store_scatter(fval, [r, cc],
                                   jnp.zeros((_L,), jnp.float32))

            n_sub = lax.shift_right_logical(cur + 127, 7)

            def sub(j, carry):
                qj = q + j

                @pl.when(j < n_sub)
                def _():
                    @pl.when(lax.rem(qj, 2) == 0)
                    def _():
                        gissue(0, qj, j)

                    @pl.when(lax.rem(qj, 2) == 1)
                    def _():
                        gissue(1, qj, j)

                @pl.when(j >= 1)
                def _():
                    qm = qj - 1

                    @pl.when(lax.rem(qm, 2) == 0)
                    def _():
                        gfinish(0, j - 1)

                    @pl.when(lax.rem(qm, 2) == 1)
                    def _():
                        gfinish(1, j - 1)

                return carry

            lax.fori_loop(0, n_sub + 1, sub, jnp.int32(0))
            return q + n_sub

        qf = lax.fori_loop(0, n_ch, chunk, jnp.int32(0))

        p1 = lax.rem(qf - 1, 2)

        def sdrain(s):
            ssm = ssem0 if s == 0 else ssem1
            pltpu.make_async_copy(emb_hbm.at[pl.ds(0, 128)], rows.at[s],
                                  ssm).wait()

        @pl.when((qf >= 1) & (p1 == 0))
        def _():
            sdrain(0)

        @pl.when((qf >= 1) & (p1 == 1))
        def _():
            sdrain(1)

        @pl.when((qf >= 2) & (p1 == 1))
        def _():
            sdrain(0)

        @pl.when((qf >= 2) & (p1 == 0))
        def _():
            sdrain(1)

        plsc.subcore_barrier()

        @pl.when(sid < wb_full)
        def _():
            pltpu.sync_copy(acc.at[pl.ds(sid * rpt, rpt)],
                            out_hbm.at[pl.ds(lo + sid * rpt, rpt)])

        if wb_rem:
            @pl.when(sid == wb_full)
            def _():
                pltpu.sync_copy(acc.at[pl.ds(wb_full * rpt, wb_rem)],
                                out_hbm.at[pl.ds(lo + wb_full * rpt, wb_rem)])

    mesh = plsc.VectorSubcoreMesh(core_axis_name="c", subcore_axis_name="s")
    return pl.kernel(
        body,
        out_type=jax.ShapeDtypeStruct((n_nodes, d), jnp.float32),
        mesh=mesh,
        compiler_params=_SC_PARAMS,
        scratch_types=[
            pltpu.VMEM((ch,), jnp.int32),
            pltpu.VMEM((ch,), jnp.int32),
            pltpu.VMEM((ch,), jnp.float32),
            pltpu.VMEM((nsb_cap, 128), jnp.int32),
            pltpu.VMEM((nsb_cap, 128), jnp.int32),
            pltpu.VMEM((nsb_cap, 128), jnp.float32),
            pltpu.VMEM((2, 128, d), jnp.float32),
            pltpu.VMEM((2, 128), jnp.int32),
            pltpu.VMEM_SHARED((acc_rows, d), jnp.float32),
            pltpu.SemaphoreType.DMA,
            pltpu.SemaphoreType.DMA,
            pltpu.SemaphoreType.DMA,
            pltpu.SemaphoreType.DMA,
            pltpu.SemaphoreType.DMA,
        ],
    )


def _combine(ae, a1, a2, a3, r1, r2, r3, w1, w2, w3, Wc, bc, Wc1, bc1):
    n, d = ae.shape
    blk = 1000
    assert n % blk == 0

    def body(aer, a1r, a2r, a3r, r1r, r2r, r3r, w1r, w2r, w3r,
             wcr, bcr, wc1r, bc1r, o):
        base = aer[...]
        hol = (base + a1r[...] + a2r[...] + a3r[...]) * 0.25
        rgt = (base + r1r[...] + r2r[...] + r3r[...]) * 0.25
        wrg = (base + w1r[...] + w2r[...] + w3r[...]) * 0.25
        wc = wcr[...]
        h = (jnp.dot(rgt, wc[:d], **_MM_KW) + jnp.dot(wrg, wc[d:], **_MM_KW)
             + bcr[...])
        dis = jnp.where(h >= 0, h, h * _LEAKY)
        wc1 = wc1r[...]
        h2 = (jnp.dot(dis, wc1[:d], **_MM_KW) + jnp.dot(hol, wc1[d:], **_MM_KW)
              + bc1r[...])
        o[...] = jnp.where(h2 >= 0, h2, h2 * _LEAKY)

    emb_spec = pl.BlockSpec((blk, d), lambda i: (i, 0))
    w_spec = pl.BlockSpec((2 * d, d), lambda i: (0, 0))
    b_spec = pl.BlockSpec((1, d), lambda i: (0, 0))
    return pl.pallas_call(
        body,
        grid=(n // blk,),
        in_specs=[emb_spec] * 10 + [w_spec, b_spec, w_spec, b_spec],
        out_specs=pl.BlockSpec((blk, d), lambda i: (i, 0)),
        out_shape=jax.ShapeDtypeStruct((n, d), jnp.float32),
    )(ae, a1, a2, a3, r1, r2, r3, w1, w2, w3,
      Wc, bc.reshape(1, d), Wc1, bc1.reshape(1, d))


def _make_batch_gather(n_nodes, d, batch, disc_n):
    per = batch // (_NC * _NS)
    assert per * _NC * _NS == batch and per % 128 == 0
    nseg = per // 128

    def body(emb_hbm, sid_hbm, eid_hbm, disc_hbm,
             stus_hbm, exers_hbm, disc_out_hbm,
             idx, rows, dvec, dout):
        core = lax.axis_index("c")
        s = lax.axis_index("s")
        w = s * _NC + core
        base = w * per

        pltpu.sync_copy(sid_hbm.at[pl.ds(base, per)], idx)
        for k in range(nseg):
            pltpu.sync_copy(emb_hbm.at[idx.at[pl.ds(k * 128, 128)]],
                            rows.at[pl.ds(k * 128, 128)])
        pltpu.sync_copy(rows, stus_hbm.at[pl.ds(base, per)])

        pltpu.sync_copy(eid_hbm.at[pl.ds(base, per)], idx)
        pltpu.sync_copy(disc_hbm, dvec)

        @pl.loop(0, per // _L)
        def _(g):
            sl = pl.ds(g * _L, _L)
            e = idx[sl]
            dout[sl] = plsc.load_gather(dvec, [e])
            idx[sl] = e + _S

        pltpu.sync_copy(dout, disc_out_hbm.at[pl.ds(base, per)])
        for k in range(nseg):
            pltpu.sync_copy(emb_hbm.at[idx.at[pl.ds(k * 128, 128)]],
                            rows.at[pl.ds(k * 128, 128)])
        pltpu.sync_copy(rows, exers_hbm.at[pl.ds(base, per)])

    mesh = plsc.VectorSubcoreMesh(core_axis_name="c", subcore_axis_name="s")
    return pl.kernel(
        body,
        out_type=(jax.ShapeDtypeStruct((batch, d), jnp.float32),
                  jax.ShapeDtypeStruct((batch, d), jnp.float32),
                  jax.ShapeDtypeStruct((batch,), jnp.float32)),
        mesh=mesh,
        compiler_params=_SC_PARAMS,
        scratch_types=[
            pltpu.VMEM((per,), jnp.int32),
            pltpu.VMEM((per, d), jnp.float32),
            pltpu.VMEM((disc_n,), jnp.float32),
            pltpu.VMEM((per,), jnp.float32),
        ],
    )


def _project(x, w, b):
    bsz, d = x.shape
    k = w.shape[1]
    blk = min(bsz, 1024)
    assert bsz % blk == 0

    def body(xr, wr, br, o):
        o[...] = jnp.dot(xr[...], wr[...], **_MM_KW) + br[...]

    return pl.pallas_call(
        body,
        grid=(bsz // blk,),
        in_specs=[pl.BlockSpec((blk, d), lambda i: (i, 0)),
                  pl.BlockSpec((d, k), lambda i: (0, 0)),
                  pl.BlockSpec((1, k), lambda i: (0, 0))],
        out_specs=pl.BlockSpec((blk, k), lambda i: (i, 0)),
        out_shape=jax.ShapeDtypeStruct((bsz, k), jnp.float32),
    )(x, w, b.reshape(1, k))


def kernel(student_id, exercise_id, q_mask, all_src, all_dst, all_val,
           right_src, right_dst, right_val, wrong_src, wrong_dst, wrong_val,
           stu_emb, exer_emb, know_emb, disc_emb, Wc, bc, Wc1, bc1,
           Wts, bts, Wte, bte, Wtk, btk):
    all_emb = jnp.concatenate([stu_emb, exer_emb, know_emb], axis=0)
    n_edges = all_src.shape[0]
    spmm = _make_spmm(_N, n_edges, _D)

    layer_outs = []
    for src, dst, val in ((all_src, all_dst, all_val),
                          (right_src, right_dst, right_val),
                          (wrong_src, wrong_dst, wrong_val)):
        src = src.astype(jnp.int32)
        dst = dst.astype(jnp.int32)
        cur = all_emb
        for _ in range(3):
            cur = spmm(src, dst, val, cur)
            layer_outs.append(cur)

    out_embs = _combine(all_emb, *layer_outs, Wc, bc, Wc1, bc1)

    gather = _make_batch_gather(_N, _D, student_id.shape[0], _E_NUM)
    stus_rows, exers_rows, disc_flat = gather(
        out_embs, student_id.astype(jnp.int32), exercise_id.astype(jnp.int32),
        disc_emb.reshape(-1))

    bsf = _project(stus_rows, Wts, bts)
    bef = _project(exers_rows, Wte, bte)
    kf = _project(out_embs[_S + _E_NUM:], Wtk, btk)
    return bsf, bef, disc_flat.reshape(-1, 1), kf
